# Initial kernel scaffold; baseline (speedup 1.0000x reference)
import jax
import jax.numpy as jnp
from jax.experimental import pallas as pl


def kernel(fs0, fs1, W0, b0, W1, b1, xi0, xi1, y):
    raise NotImplementedError("write your pallas kernel here")


# speedup vs baseline: 2.7072x; 2.7072x over previous
import functools

import jax
import jax.numpy as jnp
from jax.experimental import pallas as pl
from jax.experimental.pallas import tpu as pltpu

B, C = 256, 512
K0, K1, NC = 16384, 4096, 1000
HW = 64

LO0, NH0 = 512, 32
LO1, NH1 = 128, 32

BB = 4
BK = 1024
BN = 2048
BM = 128


def _scatter_build_kernel(fs0_ref, fs1_ref, xi0_ref, xi1_ref, f0_ref, f1_ref):
    t0 = fs0_ref[...].sum(axis=2) * (1.0 / HW)
    t1 = fs1_ref[...].sum(axis=2) * (1.0 / HW)
    xi0 = xi0_ref[0]
    xi1 = xi1_ref[0]

    ci = jax.lax.broadcasted_iota(jnp.int32, (BB, C, C), 1)
    cj = jax.lax.broadcasted_iota(jnp.int32, (BB, C, C), 2)
    later = cj > ci

    def dedup(t, xi):
        eq = xi[:, :, None] == xi[:, None, :]
        dup = jnp.sum(jnp.where(eq & later, 1.0, 0.0), axis=2)
        return jnp.where(dup == 0.0, t, 0.0)

    t0d = dedup(t0, xi0)
    t1d = dedup(t1, xi1)

    def build(t, xi, lo_width, n_hi, out_ref, out_dtype):
        hi = xi // lo_width
        lo = xi - hi * lo_width
        hid = jax.lax.broadcasted_iota(jnp.int32, (BB, n_hi, C), 1)
        u = jnp.where(hid == hi[:, None, :], t[:, None, :], 0.0)
        lod = jax.lax.broadcasted_iota(jnp.int32, (BB, C, lo_width), 2)
        p = jnp.where(lod == lo[:, :, None], 1.0, 0.0)
        for i in range(BB):
            f = jax.lax.dot_general(
                u[i], p[i], (((1,), (0,)), ((), ())),
                preferred_element_type=jnp.float32)
            out_ref[i] = f.astype(out_dtype)

    build(t0d, xi0, LO0, NH0, f0_ref, jnp.bfloat16)
    build(t1d, xi1, LO1, NH1, f1_ref, jnp.float32)


def _matmul1_kernel(f0_ref, w0_ref, b0_ref, h_ref, acc_ref):
    k = pl.program_id(1)
    nk = pl.num_programs(1)
    prod = jax.lax.dot_general(
        f0_ref[...].astype(jnp.float32), w0_ref[...],
        (((1,), (1,)), ((), ())), preferred_element_type=jnp.float32)

    @pl.when(k == 0)
    def _():
        acc_ref[...] = prod

    @pl.when(k > 0)
    def _():
        acc_ref[...] += prod

    @pl.when(k == nk - 1)
    def _():
        h_ref[...] = jax.nn.sigmoid(acc_ref[...] + b0_ref[...])


def _head_kernel(h_ref, f1_ref, w1_ref, b1_ref, y2_ref,
                 y_ref, ce_ref, be_ref):
    h = h_ref[...]
    logits = jax.lax.dot_general(
        h, w1_ref[...], (((1,), (1,)), ((), ())),
        preferred_element_type=jnp.float32) + b1_ref[...]
    ysig = jax.nn.sigmoid(logits)
    y_ref[...] = ysig

    be = jnp.log1p(jnp.exp(h)) - h * f1_ref[...]
    be_ref[0] = jnp.sum(be, axis=1, keepdims=True).sum(
        axis=0, keepdims=True)

    m = jnp.max(ysig, axis=1, keepdims=True)
    lse = jnp.log(jnp.sum(jnp.exp(ysig - m), axis=1, keepdims=True)) + m
    lane = jax.lax.broadcasted_iota(jnp.int32, (BM, NC), 1)
    picked = jnp.sum(
        jnp.where(lane == y2_ref[...][:, 0:1], ysig, 0.0),
        axis=1, keepdims=True)
    ce_ref[0] = jnp.sum(lse - picked, axis=0, keepdims=True)


@jax.jit
def kernel(fs0, fs1, W0, b0, W1, b1, xi0, xi1, y):
    fs0r = fs0.reshape(B, C, HW)
    fs1r = fs1.reshape(B, C, HW)
    xi0 = xi0.astype(jnp.int32).reshape(B // BB, BB, C)
    xi1 = xi1.astype(jnp.int32).reshape(B // BB, BB, C)

    f0_3d, f1_3d = pl.pallas_call(
        _scatter_build_kernel,
        grid=(B // BB,),
        in_specs=[
            pl.BlockSpec((BB, C, HW), lambda b: (b, 0, 0)),
            pl.BlockSpec((BB, C, HW), lambda b: (b, 0, 0)),
            pl.BlockSpec((1, BB, C), lambda b: (b, 0, 0)),
            pl.BlockSpec((1, BB, C), lambda b: (b, 0, 0)),
        ],
        out_specs=[
            pl.BlockSpec((BB, NH0, LO0), lambda b: (b, 0, 0)),
            pl.BlockSpec((BB, NH1, LO1), lambda b: (b, 0, 0)),
        ],
        out_shape=[
            jax.ShapeDtypeStruct((B, NH0, LO0), jnp.bfloat16),
            jax.ShapeDtypeStruct((B, NH1, LO1), jnp.float32),
        ],
        compiler_params=pltpu.CompilerParams(
            dimension_semantics=("parallel",),
            vmem_limit_bytes=60 * 1024 * 1024,
        ),
    )(fs0r, fs1r, xi0, xi1)

    f0 = f0_3d.reshape(B, K0)
    f1 = f1_3d.reshape(B, K1)

    h = pl.pallas_call(
        _matmul1_kernel,
        grid=(K1 // BN, K0 // BK),
        in_specs=[
            pl.BlockSpec((B, BK), lambda n, k: (0, k)),
            pl.BlockSpec((BN, BK), lambda n, k: (n, k)),
            pl.BlockSpec((1, BN), lambda n, k: (0, n)),
        ],
        out_specs=pl.BlockSpec((B, BN), lambda n, k: (0, n)),
        out_shape=jax.ShapeDtypeStruct((B, K1), jnp.float32),
        scratch_shapes=[pltpu.VMEM((B, BN), jnp.float32)],
        compiler_params=pltpu.CompilerParams(
            dimension_semantics=("parallel", "arbitrary"),
            vmem_limit_bytes=60 * 1024 * 1024,
        ),
    )(f0, W0, b0.reshape(1, K1))

    y2 = jnp.broadcast_to(y.astype(jnp.int32)[:, None], (B, 128))
    y_, ce_p, be_p = pl.pallas_call(
        _head_kernel,
        grid=(B // BM,),
        in_specs=[
            pl.BlockSpec((BM, K1), lambda m: (m, 0)),
            pl.BlockSpec((BM, K1), lambda m: (m, 0)),
            pl.BlockSpec((NC, K1), lambda m: (0, 0)),
            pl.BlockSpec((1, NC), lambda m: (0, 0)),
            pl.BlockSpec((BM, 128), lambda m: (m, 0)),
        ],
        out_specs=[
            pl.BlockSpec((BM, NC), lambda m: (m, 0)),
            pl.BlockSpec((1, 1, 1), lambda m: (m, 0, 0)),
            pl.BlockSpec((1, 1, 1), lambda m: (m, 0, 0)),
        ],
        out_shape=[
            jax.ShapeDtypeStruct((B, NC), jnp.float32),
            jax.ShapeDtypeStruct((B // BM, 1, 1), jnp.float32),
            jax.ShapeDtypeStruct((B // BM, 1, 1), jnp.float32),
        ],
        compiler_params=pltpu.CompilerParams(
            dimension_semantics=("parallel",),
            vmem_limit_bytes=60 * 1024 * 1024,
        ),
    )(h, f1, W1, b1.reshape(1, NC), y2)

    loss = ce_p.sum() / B + be_p.sum() / (B * K1)
    return y_, loss


# speedup vs baseline: 2.7657x; 1.0216x over previous
import functools

import jax
import jax.numpy as jnp
from jax.experimental import pallas as pl
from jax.experimental.pallas import tpu as pltpu

B, C = 256, 512
K0, K1, NC = 16384, 4096, 1000
HW = 64

LO0, NH0 = 512, 32
LO1, NH1 = 128, 32

BB = 8
BK = 2048
BN = 2048
BM = 128


def _scatter_build_kernel(fs0_ref, fs1_ref, xi0_ref, xi1_ref, f0_ref, f1_ref):
    t0 = fs0_ref[...].sum(axis=2) * (1.0 / HW)
    t1 = fs1_ref[...].sum(axis=2) * (1.0 / HW)
    xi0 = xi0_ref[0]
    xi1 = xi1_ref[0]

    ci = jax.lax.broadcasted_iota(jnp.int32, (BB, C, C), 1)
    cj = jax.lax.broadcasted_iota(jnp.int32, (BB, C, C), 2)
    later = cj > ci

    def dedup(t, xi):
        eq = xi[:, :, None] == xi[:, None, :]
        dup = jnp.sum(jnp.where(eq & later, 1.0, 0.0), axis=2)
        return jnp.where(dup == 0.0, t, 0.0)

    t0d = dedup(t0, xi0)
    t1d = dedup(t1, xi1)

    def build(t, xi, lo_width, n_hi, out_ref, out_dtype):
        hi = xi // lo_width
        lo = xi - hi * lo_width
        hid = jax.lax.broadcasted_iota(jnp.int32, (BB, n_hi, C), 1)
        u = jnp.where(hid == hi[:, None, :], t[:, None, :], 0.0)
        lod = jax.lax.broadcasted_iota(jnp.int32, (BB, C, lo_width), 2)
        p = jnp.where(lod == lo[:, :, None], 1.0, 0.0)
        for i in range(BB):
            f = jax.lax.dot_general(
                u[i], p[i], (((1,), (0,)), ((), ())),
                preferred_element_type=jnp.float32)
            out_ref[i] = f.astype(out_dtype)

    build(t0d, xi0, LO0, NH0, f0_ref, jnp.bfloat16)
    build(t1d, xi1, LO1, NH1, f1_ref, jnp.float32)


def _matmul1_kernel(f0_ref, w0_ref, b0_ref, h_ref, acc_ref):
    k = pl.program_id(1)
    nk = pl.num_programs(1)
    prod = jax.lax.dot_general(
        f0_ref[...].astype(jnp.float32), w0_ref[...],
        (((1,), (1,)), ((), ())), preferred_element_type=jnp.float32)

    @pl.when(k == 0)
    def _():
        acc_ref[...] = prod

    @pl.when(k > 0)
    def _():
        acc_ref[...] += prod

    @pl.when(k == nk - 1)
    def _():
        h_ref[...] = jax.nn.sigmoid(acc_ref[...] + b0_ref[...])


def _head_kernel(h_ref, f1_ref, w1_ref, b1_ref, y2_ref,
                 y_ref, ce_ref, be_ref):
    h = h_ref[...]
    logits = jax.lax.dot_general(
        h, w1_ref[...], (((1,), (1,)), ((), ())),
        preferred_element_type=jnp.float32) + b1_ref[...]
    ysig = jax.nn.sigmoid(logits)
    y_ref[...] = ysig

    be = jnp.log1p(jnp.exp(h)) - h * f1_ref[...]
    be_ref[0] = jnp.sum(be, axis=1, keepdims=True).sum(
        axis=0, keepdims=True)

    m = jnp.max(ysig, axis=1, keepdims=True)
    lse = jnp.log(jnp.sum(jnp.exp(ysig - m), axis=1, keepdims=True)) + m
    lane = jax.lax.broadcasted_iota(jnp.int32, (BM, NC), 1)
    picked = jnp.sum(
        jnp.where(lane == y2_ref[...][:, 0:1], ysig, 0.0),
        axis=1, keepdims=True)
    ce_ref[0] = jnp.sum(lse - picked, axis=0, keepdims=True)


@jax.jit
def kernel(fs0, fs1, W0, b0, W1, b1, xi0, xi1, y):
    fs0r = fs0.reshape(B, C, HW)
    fs1r = fs1.reshape(B, C, HW)
    xi0 = xi0.astype(jnp.int32).reshape(B // BB, BB, C)
    xi1 = xi1.astype(jnp.int32).reshape(B // BB, BB, C)

    f0_3d, f1_3d = pl.pallas_call(
        _scatter_build_kernel,
        grid=(B // BB,),
        in_specs=[
            pl.BlockSpec((BB, C, HW), lambda b: (b, 0, 0)),
            pl.BlockSpec((BB, C, HW), lambda b: (b, 0, 0)),
            pl.BlockSpec((1, BB, C), lambda b: (b, 0, 0)),
            pl.BlockSpec((1, BB, C), lambda b: (b, 0, 0)),
        ],
        out_specs=[
            pl.BlockSpec((BB, NH0, LO0), lambda b: (b, 0, 0)),
            pl.BlockSpec((BB, NH1, LO1), lambda b: (b, 0, 0)),
        ],
        out_shape=[
            jax.ShapeDtypeStruct((B, NH0, LO0), jnp.bfloat16),
            jax.ShapeDtypeStruct((B, NH1, LO1), jnp.float32),
        ],
        compiler_params=pltpu.CompilerParams(
            dimension_semantics=("parallel",),
            vmem_limit_bytes=60 * 1024 * 1024,
        ),
    )(fs0r, fs1r, xi0, xi1)

    f0 = f0_3d.reshape(B, K0)
    f1 = f1_3d.reshape(B, K1)

    h = pl.pallas_call(
        _matmul1_kernel,
        grid=(K1 // BN, K0 // BK),
        in_specs=[
            pl.BlockSpec((B, BK), lambda n, k: (0, k)),
            pl.BlockSpec((BN, BK), lambda n, k: (n, k)),
            pl.BlockSpec((1, BN), lambda n, k: (0, n)),
        ],
        out_specs=pl.BlockSpec((B, BN), lambda n, k: (0, n)),
        out_shape=jax.ShapeDtypeStruct((B, K1), jnp.float32),
        scratch_shapes=[pltpu.VMEM((B, BN), jnp.float32)],
        compiler_params=pltpu.CompilerParams(
            dimension_semantics=("parallel", "arbitrary"),
            vmem_limit_bytes=60 * 1024 * 1024,
        ),
    )(f0, W0, b0.reshape(1, K1))

    y2 = jnp.broadcast_to(y.astype(jnp.int32)[:, None], (B, 128))
    y_, ce_p, be_p = pl.pallas_call(
        _head_kernel,
        grid=(B // BM,),
        in_specs=[
            pl.BlockSpec((BM, K1), lambda m: (m, 0)),
            pl.BlockSpec((BM, K1), lambda m: (m, 0)),
            pl.BlockSpec((NC, K1), lambda m: (0, 0)),
            pl.BlockSpec((1, NC), lambda m: (0, 0)),
            pl.BlockSpec((BM, 128), lambda m: (m, 0)),
        ],
        out_specs=[
            pl.BlockSpec((BM, NC), lambda m: (m, 0)),
            pl.BlockSpec((1, 1, 1), lambda m: (m, 0, 0)),
            pl.BlockSpec((1, 1, 1), lambda m: (m, 0, 0)),
        ],
        out_shape=[
            jax.ShapeDtypeStruct((B, NC), jnp.float32),
            jax.ShapeDtypeStruct((B // BM, 1, 1), jnp.float32),
            jax.ShapeDtypeStruct((B // BM, 1, 1), jnp.float32),
        ],
        compiler_params=pltpu.CompilerParams(
            dimension_semantics=("parallel",),
            vmem_limit_bytes=60 * 1024 * 1024,
        ),
    )(h, f1, W1, b1.reshape(1, NC), y2)

    loss = ce_p.sum() / B + be_p.sum() / (B * K1)
    return y_, loss


# speedup vs baseline: 3.3807x; 1.2223x over previous
import functools

import jax
import jax.numpy as jnp
from jax.experimental import pallas as pl
from jax.experimental.pallas import tpu as pltpu

B, C = 256, 512
K0, K1, NC = 16384, 4096, 1000
HW = 64

LO0, NH0 = 512, 32
LO1, NH1 = 128, 32

BB = 8
BK = 2048
BN = 2048
BM = 128


def _scatter_build_kernel(fs0_ref, fs1_ref, xi0_ref, xi1_ref, f0_ref, f1_ref):
    t0 = fs0_ref[...].sum(axis=1) * (1.0 / HW)
    t1 = fs1_ref[...].sum(axis=1) * (1.0 / HW)
    xi0 = xi0_ref[0]
    xi1 = xi1_ref[0]

    ci = jax.lax.broadcasted_iota(jnp.int32, (BB, C, C), 1)
    cj = jax.lax.broadcasted_iota(jnp.int32, (BB, C, C), 2)
    later = cj > ci

    def dedup(t, xi):
        eq = xi[:, :, None] == xi[:, None, :]
        dup = jnp.sum(jnp.where(eq & later, 1.0, 0.0), axis=2)
        return jnp.where(dup == 0.0, t, 0.0)

    t0d = dedup(t0, xi0)
    t1d = dedup(t1, xi1)

    def build(t, xi, lo_width, n_hi, out_ref, out_dtype):
        hi = xi // lo_width
        lo = xi - hi * lo_width
        hid = jax.lax.broadcasted_iota(jnp.int32, (BB, n_hi, C), 1)
        u = jnp.where(hid == hi[:, None, :], t[:, None, :], 0.0)
        lod = jax.lax.broadcasted_iota(jnp.int32, (BB, C, lo_width), 2)
        p = jnp.where(lod == lo[:, :, None], 1.0, 0.0)
        for i in range(BB):
            f = jax.lax.dot_general(
                u[i], p[i], (((1,), (0,)), ((), ())),
                preferred_element_type=jnp.float32)
            out_ref[i] = f.astype(out_dtype)

    build(t0d, xi0, LO0, NH0, f0_ref, jnp.bfloat16)
    build(t1d, xi1, LO1, NH1, f1_ref, jnp.float32)


def _matmul1_kernel(f0_ref, w0_ref, b0_ref, h_ref, acc_ref):
    k = pl.program_id(1)
    nk = pl.num_programs(1)
    prod = jax.lax.dot_general(
        f0_ref[...].astype(jnp.float32), w0_ref[...],
        (((1,), (1,)), ((), ())), preferred_element_type=jnp.float32)

    @pl.when(k == 0)
    def _():
        acc_ref[...] = prod

    @pl.when(k > 0)
    def _():
        acc_ref[...] += prod

    @pl.when(k == nk - 1)
    def _():
        h_ref[...] = jax.nn.sigmoid(acc_ref[...] + b0_ref[...])


def _head_kernel(h_ref, f1_ref, w1_ref, b1_ref, y2_ref,
                 y_ref, ce_ref, be_ref):
    h = h_ref[...]
    logits = jax.lax.dot_general(
        h, w1_ref[...], (((1,), (1,)), ((), ())),
        preferred_element_type=jnp.float32) + b1_ref[...]
    ysig = jax.nn.sigmoid(logits)
    y_ref[...] = ysig

    be = jnp.log1p(jnp.exp(h)) - h * f1_ref[...]
    be_ref[0] = jnp.sum(be, axis=1, keepdims=True).sum(
        axis=0, keepdims=True)

    m = jnp.max(ysig, axis=1, keepdims=True)
    lse = jnp.log(jnp.sum(jnp.exp(ysig - m), axis=1, keepdims=True)) + m
    lane = jax.lax.broadcasted_iota(jnp.int32, (BM, NC), 1)
    picked = jnp.sum(
        jnp.where(lane == y2_ref[...][:, 0:1], ysig, 0.0),
        axis=1, keepdims=True)
    ce_ref[0] = jnp.sum(lse - picked, axis=0, keepdims=True)


@jax.jit
def kernel(fs0, fs1, W0, b0, W1, b1, xi0, xi1, y):
    fs0r = fs0.reshape(B, C, HW).transpose(0, 2, 1)
    fs1r = fs1.reshape(B, C, HW).transpose(0, 2, 1)
    xi0 = xi0.astype(jnp.int32).reshape(B // BB, BB, C)
    xi1 = xi1.astype(jnp.int32).reshape(B // BB, BB, C)

    f0_3d, f1_3d = pl.pallas_call(
        _scatter_build_kernel,
        grid=(B // BB,),
        in_specs=[
            pl.BlockSpec((BB, HW, C), lambda b: (b, 0, 0)),
            pl.BlockSpec((BB, HW, C), lambda b: (b, 0, 0)),
            pl.BlockSpec((1, BB, C), lambda b: (b, 0, 0)),
            pl.BlockSpec((1, BB, C), lambda b: (b, 0, 0)),
        ],
        out_specs=[
            pl.BlockSpec((BB, NH0, LO0), lambda b: (b, 0, 0)),
            pl.BlockSpec((BB, NH1, LO1), lambda b: (b, 0, 0)),
        ],
        out_shape=[
            jax.ShapeDtypeStruct((B, NH0, LO0), jnp.bfloat16),
            jax.ShapeDtypeStruct((B, NH1, LO1), jnp.float32),
        ],
        compiler_params=pltpu.CompilerParams(
            dimension_semantics=("parallel",),
            vmem_limit_bytes=60 * 1024 * 1024,
        ),
    )(fs0r, fs1r, xi0, xi1)

    f0 = f0_3d.reshape(B, K0)
    f1 = f1_3d.reshape(B, K1)

    h = pl.pallas_call(
        _matmul1_kernel,
        grid=(K1 // BN, K0 // BK),
        in_specs=[
            pl.BlockSpec((B, BK), lambda n, k: (0, k)),
            pl.BlockSpec((BN, BK), lambda n, k: (n, k)),
            pl.BlockSpec((1, BN), lambda n, k: (0, n)),
        ],
        out_specs=pl.BlockSpec((B, BN), lambda n, k: (0, n)),
        out_shape=jax.ShapeDtypeStruct((B, K1), jnp.float32),
        scratch_shapes=[pltpu.VMEM((B, BN), jnp.float32)],
        compiler_params=pltpu.CompilerParams(
            dimension_semantics=("parallel", "arbitrary"),
            vmem_limit_bytes=60 * 1024 * 1024,
        ),
    )(f0, W0, b0.reshape(1, K1))

    y2 = jnp.broadcast_to(y.astype(jnp.int32)[:, None], (B, 128))
    y_, ce_p, be_p = pl.pallas_call(
        _head_kernel,
        grid=(B // BM,),
        in_specs=[
            pl.BlockSpec((BM, K1), lambda m: (m, 0)),
            pl.BlockSpec((BM, K1), lambda m: (m, 0)),
            pl.BlockSpec((NC, K1), lambda m: (0, 0)),
            pl.BlockSpec((1, NC), lambda m: (0, 0)),
            pl.BlockSpec((BM, 128), lambda m: (m, 0)),
        ],
        out_specs=[
            pl.BlockSpec((BM, NC), lambda m: (m, 0)),
            pl.BlockSpec((1, 1, 1), lambda m: (m, 0, 0)),
            pl.BlockSpec((1, 1, 1), lambda m: (m, 0, 0)),
        ],
        out_shape=[
            jax.ShapeDtypeStruct((B, NC), jnp.float32),
            jax.ShapeDtypeStruct((B // BM, 1, 1), jnp.float32),
            jax.ShapeDtypeStruct((B // BM, 1, 1), jnp.float32),
        ],
        compiler_params=pltpu.CompilerParams(
            dimension_semantics=("parallel",),
            vmem_limit_bytes=60 * 1024 * 1024,
        ),
    )(h, f1, W1, b1.reshape(1, NC), y2)

    loss = ce_p.sum() / B + be_p.sum() / (B * K1)
    return y_, loss
